# baseline (speedup 1.0000x reference)
import functools

import numpy as np

import jax
import jax.numpy as jnp
from jax import lax
from jax.experimental import pallas as pl
from jax.experimental.pallas import tpu as pltpu
from jax.experimental.pallas import tpu_sc as plsc

NC = 2
NS = 16
L = 16
CH = 128
CHA = 64
D = 128

_PERM = np.empty((2 * D,), np.int32)
for _g in range(2 * D // 32):
    for _k in range(16):
        _PERM[32 * _g + 2 * _k] = 32 * _g + _k
        _PERM[32 * _g + 2 * _k + 1] = 32 * _g + 16 + _k


def _pad_to(n, m):
    return -(-n // m) * m


@functools.lru_cache(maxsize=None)
def _build_deg_kernel(nchunk, n_pad):
    cpt = nchunk // (NC * NS)
    mesh = plsc.VectorSubcoreMesh(core_axis_name="c", subcore_axis_name="s",
                                  num_cores=NC, num_subcores=NS)

    @functools.partial(
        pl.kernel,
        out_type=jax.ShapeDtypeStruct((NC * NS, n_pad), jnp.float32),
        mesh=mesh,
        scratch_types=[
            pltpu.VMEM((cpt, CH), jnp.int32),
            pltpu.VMEM((n_pad,), jnp.float32),
        ],
        compiler_params=pltpu.CompilerParams(needs_layout_passes=False),
    )
    def deg_kernel(dst_hbm, out_hbm, idx_v, hist_v):
        c = lax.axis_index("c")
        s = lax.axis_index("s")
        wid = s * NC + c
        pltpu.sync_copy(dst_hbm.at[pl.ds(wid * cpt, cpt)], idx_v)
        zero = jnp.zeros((L,), jnp.float32)

        @pl.loop(0, n_pad // L)
        def _(j):
            hist_v[pl.ds(j * L, L)] = zero

        ones = jnp.ones((L,), jnp.float32)

        @pl.loop(0, cpt)
        def _(r):
            for k in range(CH // L):
                idx = idx_v[r, pl.ds(k * L, L)]
                plsc.addupdate_scatter(hist_v, [idx], ones)

        pltpu.sync_copy(hist_v, out_hbm.at[wid])

    return deg_kernel


@functools.lru_cache(maxsize=None)
def _build_acc_kernel(nchunk, n_pad):
    cpt = nchunk // NS
    rpt = n_pad // NS
    mesh = plsc.VectorSubcoreMesh(core_axis_name="c", subcore_axis_name="s",
                                  num_cores=NC, num_subcores=NS)

    nbi = 4

    @functools.partial(
        pl.kernel,
        out_type=jax.ShapeDtypeStruct((NC, n_pad, D), jnp.float32),
        mesh=mesh,
        scratch_types=[
            pltpu.VMEM((nbi, 2, CHA), jnp.int32),
            pltpu.VMEM((2, CHA, D), jnp.bfloat16),
            pltpu.VMEM((2, CHA, D), jnp.float32),
            pltpu.VMEM_SHARED((n_pad, D), jnp.float32),
        ] + [pltpu.SemaphoreType.DMA] * (4 + nbi),
        compiler_params=pltpu.CompilerParams(use_tc_tiling_on_sc=False,
                                             needs_layout_passes=False),
    )
    def acc_kernel(g_hbm, idx_hbm, out_hbm, idx_v, brow_v, rows_v, acc_sh,
                   *sems):
        c = lax.axis_index("c")
        s = lax.axis_index("s")
        gbase = s * cpt
        semg = sems[:2]
        semsc = sems[2:4]
        semi = sems[4:]
        mask_hi = jnp.full((L,), -65536, jnp.int32)

        def convert(b):
            @pl.loop(0, CHA)
            def _(r):
                for gix in range(D // 32):
                    w = plsc.bitcast(brow_v[b, r, pl.ds(gix * 32, 32)],
                                     jnp.int32)
                    lo = plsc.bitcast(lax.shift_left(w, 16), jnp.float32)
                    hi = plsc.bitcast(lax.bitwise_and(w, mask_hi),
                                      jnp.float32)
                    rows_v[b, r, pl.ds(gix * 32, L)] = lo
                    rows_v[b, r, pl.ds(gix * 32 + L, L)] = hi

        @pl.loop(0, rpt // CHA)
        def _(i):
            base = s * rpt + i * CHA
            pltpu.sync_copy(g_hbm.at[pl.ds(c * n_pad + base, CHA)],
                            brow_v.at[0])
            convert(0)
            pltpu.sync_copy(rows_v.at[0], acc_sh.at[pl.ds(base, CHA)])

        plsc.subcore_barrier()

        for k in range(3):
            pltpu.async_copy(idx_hbm.at[c, gbase + k], idx_v.at[k], semi[k])
        pltpu.make_async_copy(idx_hbm.at[c, 0], idx_v.at[0], semi[0]).wait()
        pltpu.async_copy(g_hbm.at[idx_v.at[0, 0]], brow_v.at[0], semg[0])

        @pl.loop(0, cpt, step=nbi)
        def _(j0):
            for i in range(nbi):
                j = j0 + i
                b = i % 2
                bn = 1 - b

                @pl.when(j >= 1)
                def _():
                    pltpu.make_async_copy(g_hbm.at[pl.ds(0, CHA)],
                                          rows_v.at[bn], semsc[bn]).wait()

                @pl.when(j + 1 < cpt)
                def _():
                    pltpu.make_async_copy(idx_hbm.at[c, 0],
                                          idx_v.at[(i + 1) % nbi],
                                          semi[(i + 1) % nbi]).wait()
                    pltpu.async_copy(g_hbm.at[idx_v.at[(i + 1) % nbi, 0]],
                                     brow_v.at[bn], semg[bn])

                @pl.when(j + 3 < cpt)
                def _():
                    pltpu.async_copy(idx_hbm.at[c, gbase + j + 3],
                                     idx_v.at[(i + 3) % nbi],
                                     semi[(i + 3) % nbi])

                pltpu.make_async_copy(g_hbm.at[pl.ds(0, CHA)],
                                      brow_v.at[b], semg[b]).wait()
                convert(b)
                pltpu.async_copy(rows_v.at[b], acc_sh.at[idx_v.at[i, 1]],
                                 semsc[b], add=True)

        pltpu.make_async_copy(g_hbm.at[pl.ds(0, CHA)],
                              rows_v.at[(cpt - 1) % 2],
                              semsc[(cpt - 1) % 2]).wait()

        plsc.subcore_barrier()

        @pl.loop(0, rpt // CHA)
        def _(i):
            base = s * rpt + i * CHA
            pltpu.sync_copy(acc_sh.at[pl.ds(base, CHA)], rows_v.at[0])
            pltpu.sync_copy(rows_v.at[0], out_hbm.at[c, pl.ds(base, CHA)])

    return acc_kernel


def _mm_body(x_ref, w_ref, deg_ref, o_ref, ds_ref):
    h = jnp.dot(x_ref[...], w_ref[...], preferred_element_type=jnp.float32)
    deg = jnp.sum(deg_ref[...], axis=0)[:, None] + 1.0
    ds_ref[...] = deg
    dis = lax.rsqrt(deg)
    g = (h * dis).astype(jnp.bfloat16)
    o_ref[0] = g[:, :D]
    o_ref[1] = g[:, D:]


def _fin_body(amu_ref, alv_ref, deg_ref, eps_ref, z_ref, mu_ref, lv_ref):
    dis = lax.rsqrt(deg_ref[...])
    mu = amu_ref[0] * dis
    lv = alv_ref[0] * dis
    mu_ref[...] = mu
    lv_ref[...] = lv
    z_ref[...] = mu + jnp.exp(0.5 * lv) * eps_ref[...]


def kernel(x, edge_index, W_mu, W_log_var):
    N, DIN = x.shape
    E = edge_index.shape[1]
    n_pad = _pad_to(N, NS * CH)
    if n_pad < N + 1:
        n_pad += NS * CH
    e_pad = _pad_to(E, NC * NS * CH * 8)
    nchunk = e_pad // CH
    nchunka = e_pad // CHA

    src = edge_index[0].astype(jnp.int32)
    dst = edge_index[1].astype(jnp.int32)
    pad = e_pad - E
    src_p = jnp.concatenate([src, jnp.zeros((pad,), jnp.int32)]).reshape(nchunk, CH)
    dst_p = jnp.concatenate([dst, jnp.full((pad,), N, jnp.int32)]).reshape(nchunk, CH)
    offs = jnp.arange(NC, dtype=jnp.int32) * n_pad
    src_a = src_p.reshape(nchunka, CHA)
    dst_a = dst_p.reshape(nchunka, CHA)
    src2 = src_a[None] + offs[:, None, None]
    idx_cat = jnp.stack(
        [src2, jnp.broadcast_to(dst_a[None], src2.shape)], axis=2)

    deg_t = _build_deg_kernel(nchunk, n_pad)(dst_p)

    xp = jnp.pad(x.astype(jnp.float32), ((0, n_pad - N), (0, 0)))
    w_cat = jnp.concatenate([W_mu, W_log_var], axis=1)[:, _PERM]

    bm = 512
    g = pl.pallas_call(
        _mm_body,
        grid=(n_pad // bm,),
        in_specs=[
            pl.BlockSpec((bm, DIN), lambda i: (i, 0)),
            pl.BlockSpec((DIN, 2 * D), lambda i: (0, 0)),
            pl.BlockSpec((NC * NS, bm), lambda i: (0, i)),
        ],
        out_specs=[
            pl.BlockSpec((NC, bm, D), lambda i: (0, i, 0)),
            pl.BlockSpec((bm, 1), lambda i: (i, 0)),
        ],
        out_shape=[
            jax.ShapeDtypeStruct((NC, n_pad, D), jnp.bfloat16),
            jax.ShapeDtypeStruct((n_pad, 1), jnp.float32),
        ],
    )(xp, w_cat, deg_t)
    g, degsum = g
    g_flat = g.reshape(NC * n_pad, D)

    acc = _build_acc_kernel(nchunka, n_pad)(g_flat, idx_cat)

    eps = jax.random.normal(jax.random.key(1), (N, D), jnp.float32)

    bf = 400
    z, mu, lv = pl.pallas_call(
        _fin_body,
        grid=(N // bf,),
        in_specs=[
            pl.BlockSpec((1, bf, D), lambda i: (0, i, 0)),
            pl.BlockSpec((1, bf, D), lambda i: (1, i, 0)),
            pl.BlockSpec((bf, 1), lambda i: (i, 0)),
            pl.BlockSpec((bf, D), lambda i: (i, 0)),
        ],
        out_specs=[
            pl.BlockSpec((bf, D), lambda i: (i, 0)),
            pl.BlockSpec((bf, D), lambda i: (i, 0)),
            pl.BlockSpec((bf, D), lambda i: (i, 0)),
        ],
        out_shape=[
            jax.ShapeDtypeStruct((N, D), jnp.float32),
            jax.ShapeDtypeStruct((N, D), jnp.float32),
            jax.ShapeDtypeStruct((N, D), jnp.float32),
        ],
    )(acc, acc, degsum, eps)
    return (z, mu, lv)


# speedup vs baseline: 1.0591x; 1.0591x over previous
import functools

import jax
import jax.numpy as jnp
from jax import lax
from jax.experimental import pallas as pl
from jax.experimental.pallas import tpu as pltpu
from jax.experimental.pallas import tpu_sc as plsc

NC = 2
NS = 16
L = 16
CH = 128
CHA = 64
D = 128


def _pad_to(n, m):
    return -(-n // m) * m


@functools.lru_cache(maxsize=None)
def _build_deg_kernel(nchunk, n_pad):
    cpt = nchunk // (NC * NS)
    mesh = plsc.VectorSubcoreMesh(core_axis_name="c", subcore_axis_name="s",
                                  num_cores=NC, num_subcores=NS)

    @functools.partial(
        pl.kernel,
        out_type=jax.ShapeDtypeStruct((NC * NS, n_pad), jnp.float32),
        mesh=mesh,
        scratch_types=[
            pltpu.VMEM((cpt, CH), jnp.int32),
            pltpu.VMEM((n_pad,), jnp.float32),
        ],
        compiler_params=pltpu.CompilerParams(needs_layout_passes=False),
    )
    def deg_kernel(dst_hbm, out_hbm, idx_v, hist_v):
        c = lax.axis_index("c")
        s = lax.axis_index("s")
        wid = s * NC + c
        pltpu.sync_copy(dst_hbm.at[pl.ds(wid * cpt, cpt)], idx_v)
        zero = jnp.zeros((L,), jnp.float32)

        @pl.loop(0, n_pad // L)
        def _(j):
            hist_v[pl.ds(j * L, L)] = zero

        ones = jnp.ones((L,), jnp.float32)

        @pl.loop(0, cpt)
        def _(r):
            for k in range(CH // L):
                idx = idx_v[r, pl.ds(k * L, L)]
                plsc.addupdate_scatter(hist_v, [idx], ones)

        pltpu.sync_copy(hist_v, out_hbm.at[wid])

    return deg_kernel


@functools.lru_cache(maxsize=None)
def _build_acc_kernel(nchunk, n_pad):
    cpt = nchunk // NS
    rpt = n_pad // NS
    mesh = plsc.VectorSubcoreMesh(core_axis_name="c", subcore_axis_name="s",
                                  num_cores=NC, num_subcores=NS)

    nbr = 4
    nbi = 8
    assert cpt % nbi == 0

    @functools.partial(
        pl.kernel,
        out_type=jax.ShapeDtypeStruct((NC, n_pad, D), jnp.float32),
        mesh=mesh,
        scratch_types=[
            pltpu.VMEM((nbi, 2, CHA), jnp.int32),
            pltpu.VMEM((nbr, CHA, D), jnp.float32),
            pltpu.VMEM_SHARED((n_pad, D), jnp.float32),
        ] + [pltpu.SemaphoreType.DMA] * (nbr + nbi),
        compiler_params=pltpu.CompilerParams(use_tc_tiling_on_sc=False),
    )
    def acc_kernel(g_hbm, idx_hbm, out_hbm, idx_v, rows_v, acc_sh, *sems):
        c = lax.axis_index("c")
        s = lax.axis_index("s")
        gbase = s * cpt
        semr = sems[:nbr]
        semi = sems[nbr:]

        @pl.loop(0, rpt // CHA)
        def _(i):
            base = s * rpt + i * CHA
            pltpu.sync_copy(g_hbm.at[pl.ds(c * n_pad + base, CHA)],
                            rows_v.at[0])
            pltpu.sync_copy(rows_v.at[0], acc_sh.at[pl.ds(base, CHA)])

        plsc.subcore_barrier()

        for k in range(5):
            pltpu.async_copy(idx_hbm.at[c, gbase + k], idx_v.at[k], semi[k])
        for k in range(2):
            pltpu.make_async_copy(idx_hbm.at[c, 0], idx_v.at[k],
                                  semi[k]).wait()
            pltpu.async_copy(g_hbm.at[idx_v.at[k, 0]], rows_v.at[k],
                             semr[k])

        @pl.loop(0, cpt, step=nbi)
        def _(j0):
            for i in range(nbi):
                j = j0 + i
                b = i % nbr
                b2 = (i + 2) % nbr

                @pl.when(j >= 2)
                def _():
                    pltpu.make_async_copy(g_hbm.at[pl.ds(0, CHA)],
                                          rows_v.at[b2], semr[b2]).wait()

                @pl.when(j + 2 < cpt)
                def _():
                    pltpu.make_async_copy(idx_hbm.at[c, 0],
                                          idx_v.at[(i + 2) % nbi],
                                          semi[(i + 2) % nbi]).wait()
                    pltpu.async_copy(g_hbm.at[idx_v.at[(i + 2) % nbi, 0]],
                                     rows_v.at[b2], semr[b2])

                @pl.when(j + 5 < cpt)
                def _():
                    pltpu.async_copy(idx_hbm.at[c, gbase + j + 5],
                                     idx_v.at[(i + 5) % nbi],
                                     semi[(i + 5) % nbi])

                pltpu.make_async_copy(g_hbm.at[pl.ds(0, CHA)], rows_v.at[b],
                                      semr[b]).wait()
                pltpu.async_copy(rows_v.at[b], acc_sh.at[idx_v.at[i, 1]],
                                 semr[b], add=True)

        for k in range(2):
            b = (cpt - 2 + k) % nbr
            pltpu.make_async_copy(g_hbm.at[pl.ds(0, CHA)], rows_v.at[b],
                                  semr[b]).wait()

        plsc.subcore_barrier()

        @pl.loop(0, rpt // CHA)
        def _(i):
            base = s * rpt + i * CHA
            pltpu.sync_copy(acc_sh.at[pl.ds(base, CHA)], rows_v.at[0])
            pltpu.sync_copy(rows_v.at[0], out_hbm.at[c, pl.ds(base, CHA)])

    return acc_kernel


def _mm_body(x_ref, w_ref, deg_ref, o_ref, ds_ref):
    h = jnp.dot(x_ref[...], w_ref[...], preferred_element_type=jnp.float32)
    deg = jnp.sum(deg_ref[...], axis=0)[:, None] + 1.0
    ds_ref[...] = deg
    dis = lax.rsqrt(deg)
    g = h * dis
    o_ref[0] = g[:, :D]
    o_ref[1] = g[:, D:]


def _fin_body(amu_ref, alv_ref, deg_ref, eps_ref, z_ref, mu_ref, lv_ref):
    dis = lax.rsqrt(deg_ref[...])
    mu = amu_ref[0] * dis
    lv = alv_ref[0] * dis
    mu_ref[...] = mu
    lv_ref[...] = lv
    z_ref[...] = mu + jnp.exp(0.5 * lv) * eps_ref[...]


def kernel(x, edge_index, W_mu, W_log_var):
    N, DIN = x.shape
    E = edge_index.shape[1]
    n_pad = _pad_to(N, NS * CH)
    if n_pad < N + 1:
        n_pad += NS * CH
    e_pad = _pad_to(E, NC * NS * CH * 8)
    nchunk = e_pad // CH
    nchunka = e_pad // CHA

    src = edge_index[0].astype(jnp.int32)
    dst = edge_index[1].astype(jnp.int32)
    pad = e_pad - E
    src_p = jnp.concatenate([src, jnp.zeros((pad,), jnp.int32)]).reshape(nchunk, CH)
    dst_p = jnp.concatenate([dst, jnp.full((pad,), N, jnp.int32)]).reshape(nchunk, CH)
    offs = jnp.arange(NC, dtype=jnp.int32) * n_pad
    src_a = src_p.reshape(nchunka, CHA)
    dst_a = dst_p.reshape(nchunka, CHA)
    src2 = src_a[None] + offs[:, None, None]
    idx_cat = jnp.stack(
        [src2, jnp.broadcast_to(dst_a[None], src2.shape)], axis=2)

    deg_t = _build_deg_kernel(nchunk, n_pad)(dst_p)

    xp = jnp.pad(x.astype(jnp.float32), ((0, n_pad - N), (0, 0)))
    w_cat = jnp.concatenate([W_mu, W_log_var], axis=1)

    bm = 512
    g = pl.pallas_call(
        _mm_body,
        grid=(n_pad // bm,),
        in_specs=[
            pl.BlockSpec((bm, DIN), lambda i: (i, 0)),
            pl.BlockSpec((DIN, 2 * D), lambda i: (0, 0)),
            pl.BlockSpec((NC * NS, bm), lambda i: (0, i)),
        ],
        out_specs=[
            pl.BlockSpec((NC, bm, D), lambda i: (0, i, 0)),
            pl.BlockSpec((bm, 1), lambda i: (i, 0)),
        ],
        out_shape=[
            jax.ShapeDtypeStruct((NC, n_pad, D), jnp.float32),
            jax.ShapeDtypeStruct((n_pad, 1), jnp.float32),
        ],
    )(xp, w_cat, deg_t)
    g, degsum = g
    g_flat = g.reshape(NC * n_pad, D)

    acc = _build_acc_kernel(nchunka, n_pad)(g_flat, idx_cat)

    eps = jax.random.normal(jax.random.key(1), (N, D), jnp.float32)

    bf = 400
    z, mu, lv = pl.pallas_call(
        _fin_body,
        grid=(N // bf,),
        in_specs=[
            pl.BlockSpec((1, bf, D), lambda i: (0, i, 0)),
            pl.BlockSpec((1, bf, D), lambda i: (1, i, 0)),
            pl.BlockSpec((bf, 1), lambda i: (i, 0)),
            pl.BlockSpec((bf, D), lambda i: (i, 0)),
        ],
        out_specs=[
            pl.BlockSpec((bf, D), lambda i: (i, 0)),
            pl.BlockSpec((bf, D), lambda i: (i, 0)),
            pl.BlockSpec((bf, D), lambda i: (i, 0)),
        ],
        out_shape=[
            jax.ShapeDtypeStruct((N, D), jnp.float32),
            jax.ShapeDtypeStruct((N, D), jnp.float32),
            jax.ShapeDtypeStruct((N, D), jnp.float32),
        ],
    )(acc, acc, degsum, eps)
    return (z, mu, lv)


# speedup vs baseline: 1.0898x; 1.0290x over previous
import functools

import jax
import jax.numpy as jnp
from jax import lax
from jax.experimental import pallas as pl
from jax.experimental.pallas import tpu as pltpu
from jax.experimental.pallas import tpu_sc as plsc

NC = 2
NS = 16
L = 16
CH = 128
D = 128


def _pad_to(n, m):
    return -(-n // m) * m


@functools.lru_cache(maxsize=None)
def _build_deg_kernel(nchunk, n_pad):
    cpt = nchunk // (NC * NS)
    mesh = plsc.VectorSubcoreMesh(core_axis_name="c", subcore_axis_name="s",
                                  num_cores=NC, num_subcores=NS)

    @functools.partial(
        pl.kernel,
        out_type=jax.ShapeDtypeStruct((NC * NS, n_pad), jnp.float32),
        mesh=mesh,
        scratch_types=[
            pltpu.VMEM((cpt, CH), jnp.int32),
            pltpu.VMEM((n_pad,), jnp.float32),
        ],
        compiler_params=pltpu.CompilerParams(needs_layout_passes=False),
    )
    def deg_kernel(dst_hbm, out_hbm, idx_v, hist_v):
        c = lax.axis_index("c")
        s = lax.axis_index("s")
        wid = s * NC + c
        pltpu.sync_copy(dst_hbm.at[pl.ds(wid * cpt, cpt)], idx_v)
        zero = jnp.zeros((L,), jnp.float32)

        @pl.loop(0, n_pad // L)
        def _(j):
            hist_v[pl.ds(j * L, L)] = zero

        ones = jnp.ones((L,), jnp.float32)

        @pl.loop(0, cpt)
        def _(r):
            for k in range(CH // L):
                idx = idx_v[r, pl.ds(k * L, L)]
                plsc.addupdate_scatter(hist_v, [idx], ones)

        pltpu.sync_copy(hist_v, out_hbm.at[wid])

    return deg_kernel


@functools.lru_cache(maxsize=None)
def _build_acc_kernel(nchunk, n_pad):
    cpt = nchunk // NS
    rpt = n_pad // NS
    mesh = plsc.VectorSubcoreMesh(core_axis_name="c", subcore_axis_name="s",
                                  num_cores=NC, num_subcores=NS)

    nbi = 4

    @functools.partial(
        pl.kernel,
        out_type=jax.ShapeDtypeStruct((NC, n_pad, D), jnp.float32),
        mesh=mesh,
        scratch_types=[
            pltpu.VMEM((nbi, 2, CH), jnp.int32),
            pltpu.VMEM((2, CH, D), jnp.float32),
            pltpu.VMEM_SHARED((n_pad, D), jnp.float32),
        ] + [pltpu.SemaphoreType.DMA] * (2 + nbi),
        compiler_params=pltpu.CompilerParams(use_tc_tiling_on_sc=False),
    )
    def acc_kernel(g_hbm, idx_hbm, out_hbm, idx_v, rows_v, acc_sh, *sems):
        c = lax.axis_index("c")
        s = lax.axis_index("s")
        gbase = s * cpt
        semr = sems[:2]
        semi = sems[2:]

        @pl.loop(0, rpt // CH)
        def _(i):
            base = s * rpt + i * CH
            pltpu.sync_copy(g_hbm.at[pl.ds(c * n_pad + base, CH)],
                            rows_v.at[0])
            pltpu.sync_copy(rows_v.at[0], acc_sh.at[pl.ds(base, CH)])

        plsc.subcore_barrier()

        for k in range(3):
            pltpu.async_copy(idx_hbm.at[c, gbase + k], idx_v.at[k], semi[k])
        pltpu.make_async_copy(idx_hbm.at[c, 0], idx_v.at[0], semi[0]).wait()
        pltpu.async_copy(g_hbm.at[idx_v.at[0, 0]], rows_v.at[0], semr[0])

        @pl.loop(0, cpt, step=nbi)
        def _(j0):
            for i in range(nbi):
                j = j0 + i
                b = i % 2
                bn = 1 - b

                @pl.when(j >= 1)
                def _():
                    pltpu.make_async_copy(g_hbm.at[pl.ds(0, CH)],
                                          rows_v.at[bn], semr[bn]).wait()

                @pl.when(j + 1 < cpt)
                def _():
                    pltpu.make_async_copy(idx_hbm.at[c, 0],
                                          idx_v.at[(i + 1) % nbi],
                                          semi[(i + 1) % nbi]).wait()
                    pltpu.async_copy(g_hbm.at[idx_v.at[(i + 1) % nbi, 0]],
                                     rows_v.at[bn], semr[bn])

                @pl.when(j + 3 < cpt)
                def _():
                    pltpu.async_copy(idx_hbm.at[c, gbase + j + 3],
                                     idx_v.at[(i + 3) % nbi],
                                     semi[(i + 3) % nbi])

                pltpu.make_async_copy(g_hbm.at[pl.ds(0, CH)], rows_v.at[b],
                                      semr[b]).wait()
                pltpu.async_copy(rows_v.at[b], acc_sh.at[idx_v.at[i, 1]],
                                 semr[b], add=True)

        pltpu.make_async_copy(g_hbm.at[pl.ds(0, CH)],
                              rows_v.at[(cpt - 1) % 2],
                              semr[(cpt - 1) % 2]).wait()

        plsc.subcore_barrier()

        @pl.loop(0, rpt // CH)
        def _(i):
            base = s * rpt + i * CH
            pltpu.sync_copy(acc_sh.at[pl.ds(base, CH)], rows_v.at[0])
            pltpu.sync_copy(rows_v.at[0], out_hbm.at[c, pl.ds(base, CH)])

    return acc_kernel


def _mm_body(x_ref, w_ref, deg_ref, o_ref, ds_ref):
    h = jnp.dot(x_ref[...], w_ref[...], preferred_element_type=jnp.float32)
    deg = jnp.sum(deg_ref[...], axis=0)[:, None] + 1.0
    ds_ref[...] = deg
    dis = lax.rsqrt(deg)
    g = h * dis
    o_ref[0] = g[:, :D]
    o_ref[1] = g[:, D:]


def _fin_body(amu_ref, alv_ref, deg_ref, eps_ref, z_ref, mu_ref, lv_ref):
    dis = lax.rsqrt(deg_ref[...])
    mu = amu_ref[0] * dis
    lv = alv_ref[0] * dis
    mu_ref[...] = mu
    lv_ref[...] = lv
    z_ref[...] = mu + jnp.exp(0.5 * lv) * eps_ref[...]


def kernel(x, edge_index, W_mu, W_log_var):
    N, DIN = x.shape
    E = edge_index.shape[1]
    n_pad = _pad_to(N, NS * CH)
    if n_pad < N + 1:
        n_pad += NS * CH
    e_pad = _pad_to(E, NC * NS * CH * 8)
    nchunk = e_pad // CH

    src = edge_index[0].astype(jnp.int32)
    dst = edge_index[1].astype(jnp.int32)
    pad = e_pad - E
    src_p = jnp.concatenate([src, jnp.zeros((pad,), jnp.int32)]).reshape(nchunk, CH)
    dst_p = jnp.concatenate([dst, jnp.full((pad,), N, jnp.int32)]).reshape(nchunk, CH)
    offs = jnp.arange(NC, dtype=jnp.int32) * n_pad
    src2 = src_p[None] + offs[:, None, None]
    idx_cat = jnp.stack(
        [src2, jnp.broadcast_to(dst_p[None], src2.shape)], axis=2)

    deg_t = _build_deg_kernel(nchunk, n_pad)(dst_p)

    xp = jnp.pad(x.astype(jnp.float32), ((0, n_pad - N), (0, 0)))
    w_cat = jnp.concatenate([W_mu, W_log_var], axis=1)

    bm = 512
    g = pl.pallas_call(
        _mm_body,
        grid=(n_pad // bm,),
        in_specs=[
            pl.BlockSpec((bm, DIN), lambda i: (i, 0)),
            pl.BlockSpec((DIN, 2 * D), lambda i: (0, 0)),
            pl.BlockSpec((NC * NS, bm), lambda i: (0, i)),
        ],
        out_specs=[
            pl.BlockSpec((NC, bm, D), lambda i: (0, i, 0)),
            pl.BlockSpec((bm, 1), lambda i: (i, 0)),
        ],
        out_shape=[
            jax.ShapeDtypeStruct((NC, n_pad, D), jnp.float32),
            jax.ShapeDtypeStruct((n_pad, 1), jnp.float32),
        ],
    )(xp, w_cat, deg_t)
    g, degsum = g
    g_flat = g.reshape(NC * n_pad, D)

    acc = _build_acc_kernel(nchunk, n_pad)(g_flat, idx_cat)

    eps = jax.random.normal(jax.random.key(1), (N, D), jnp.float32)

    bf = 400
    z, mu, lv = pl.pallas_call(
        _fin_body,
        grid=(N // bf,),
        in_specs=[
            pl.BlockSpec((1, bf, D), lambda i: (0, i, 0)),
            pl.BlockSpec((1, bf, D), lambda i: (1, i, 0)),
            pl.BlockSpec((bf, 1), lambda i: (i, 0)),
            pl.BlockSpec((bf, D), lambda i: (i, 0)),
        ],
        out_specs=[
            pl.BlockSpec((bf, D), lambda i: (i, 0)),
            pl.BlockSpec((bf, D), lambda i: (i, 0)),
            pl.BlockSpec((bf, D), lambda i: (i, 0)),
        ],
        out_shape=[
            jax.ShapeDtypeStruct((N, D), jnp.float32),
            jax.ShapeDtypeStruct((N, D), jnp.float32),
            jax.ShapeDtypeStruct((N, D), jnp.float32),
        ],
    )(acc, acc, degsum, eps)
    return (z, mu, lv)
